# Initial kernel scaffold; baseline (speedup 1.0000x reference)
import jax
import jax.numpy as jnp
from jax.experimental import pallas as pl


def kernel(codes, codebooks, scales):
    raise NotImplementedError("write your pallas kernel here")


# speedup vs baseline: 68.8299x; 68.8299x over previous
import functools

import jax
import jax.numpy as jnp
from jax import lax
from jax.experimental import pallas as pl
from jax.experimental.pallas import tpu as pltpu
from jax.experimental.pallas import tpu_sc as plsc


def _make_sc_kernel(num_out, num_in_elems, flat_cb_len):
    info = plsc.get_sparse_core_info()
    nc, ns, L = info.num_cores, info.num_subcores, info.num_lanes
    nw = nc * ns
    rows_per_w = num_out // nw
    iters = num_in_elems // L

    mesh = plsc.VectorSubcoreMesh(core_axis_name="c", subcore_axis_name="s")

    @functools.partial(
        pl.kernel,
        mesh=mesh,
        out_type=jax.ShapeDtypeStruct((num_out, num_in_elems), jnp.float32),
        scratch_types=[
            pltpu.VMEM((flat_cb_len,), jnp.float32),
            pltpu.VMEM((rows_per_w,), jnp.float32),
            pltpu.VMEM((num_in_elems,), jnp.int32),
            pltpu.VMEM((num_in_elems,), jnp.float32),
        ],
        compiler_params=pltpu.CompilerParams(needs_layout_passes=False),
    )
    def k(codes_hbm, cb_hbm, scales_hbm, out_hbm, cb_v, sc_v, codes_v, out_v):
        wid = lax.axis_index("s") * nc + lax.axis_index("c")
        row0 = wid * rows_per_w
        pltpu.sync_copy(cb_hbm, cb_v)
        pltpu.sync_copy(scales_hbm.at[pl.ds(row0, rows_per_w)], sc_v)

        lane = lax.iota(jnp.int32, L)
        j_lane = lane & 7
        hi8 = lane & 8

        def row_body(r, carry):
            row = row0 + r
            pltpu.sync_copy(codes_hbm.at[row], codes_v)
            s = plsc.load_gather(sc_v, [jnp.full((L,), r, jnp.int32)])

            def it_body(it, c2):
                base = it * 16
                acc = jnp.zeros((L,), jnp.float32)
                for m in range(8):
                    cidx = base + hi8 + m
                    cvec = plsc.load_gather(codes_v, [cidx])
                    fidx = (cvec << 3) + (j_lane + m * 2048)
                    acc = acc + plsc.load_gather(cb_v, [fidx])
                out_v[pl.ds(base, L)] = acc * s
                return c2

            lax.fori_loop(0, iters, it_body, 0, unroll=4)
            pltpu.sync_copy(out_v, out_hbm.at[row])
            return carry

        lax.fori_loop(0, rows_per_w, row_body, 0)

    return k


def kernel(codes, codebooks, scales):
    num_out, num_in, num_cb = codes.shape
    _, cb_size, ogs, igs = codebooks.shape
    codes2d = codes.reshape(num_out, num_in * num_cb)
    flat_cb = codebooks.reshape(num_cb * cb_size * ogs * igs)
    scales1d = scales.reshape(num_out)
    k = _make_sc_kernel(num_out, num_in * igs, flat_cb.shape[0])
    return k(codes2d, flat_cb, scales1d)
